# Initial kernel scaffold; baseline (speedup 1.0000x reference)
import jax
import jax.numpy as jnp
from jax.experimental import pallas as pl


def kernel(x_seq, edge_index, W_gcn, b_gcn, W_ih, W_hh, b_ih, b_hh, W_cls, b_cls):
    raise NotImplementedError("write your pallas kernel here")


# speedup vs baseline: 1.0001x; 1.0001x over previous
import functools

import jax
import jax.numpy as jnp
from jax.experimental import pallas as pl
from jax.experimental.pallas import tpu as pltpu

N = 10000
E = 320000
F_IN = 128
H = 128
B = 2
T = 8


def _lstm_head_body(emb_ref, wih_t_ref, whh_t_ref, b_ref, wcls_ref, bcls_ref,
                    out_ref):
    wih_t = wih_t_ref[...]
    whh_t = whh_t_ref[...]
    b = b_ref[...]

    def step(t, hc):
        h, c = hc
        x_t = emb_ref[t]
        gates = (jnp.dot(x_t, wih_t, preferred_element_type=jnp.float32)
                 + jnp.dot(h, whh_t, preferred_element_type=jnp.float32) + b)
        i = jax.nn.sigmoid(gates[:, 0 * H:1 * H])
        f = jax.nn.sigmoid(gates[:, 1 * H:2 * H])
        g = jnp.tanh(gates[:, 2 * H:3 * H])
        o = jax.nn.sigmoid(gates[:, 3 * H:4 * H])
        c_new = f * c + i * g
        h_new = o * jnp.tanh(c_new)
        return (h_new, c_new)

    h0 = jnp.zeros((B, H), jnp.float32)
    c0 = jnp.zeros((B, H), jnp.float32)
    h, c = jax.lax.fori_loop(0, T, step, (h0, c0))
    out_ref[...] = jax.nn.sigmoid(
        jnp.dot(h, wcls_ref[...], preferred_element_type=jnp.float32)
        + bcls_ref[...])


@jax.jit
def _lstm_head(emb_tbh, W_ih, W_hh, b_ih, b_hh, W_cls, b_cls):
    b = (b_ih + b_hh).reshape(1, 4 * H)
    return pl.pallas_call(
        _lstm_head_body,
        out_shape=jax.ShapeDtypeStruct((B, 1), jnp.float32),
    )(emb_tbh, W_ih.T, W_hh.T, b, W_cls, b_cls.reshape(1, 1))


def kernel(x_seq, edge_index, W_gcn, b_gcn, W_ih, W_hh, b_ih, b_hh, W_cls,
           b_cls):
    loop = jnp.arange(N, dtype=edge_index.dtype)
    src = jnp.concatenate([edge_index[0], loop])
    dst = jnp.concatenate([edge_index[1], loop])
    deg = jnp.zeros((N,), jnp.float32).at[dst].add(1.0)
    dinv = 1.0 / jnp.sqrt(deg)
    norm = dinv[src] * dinv[dst]

    def gcn(x_t):
        h = x_t @ W_gcn
        msg = h[src] * norm[:, None]
        agg = jax.ops.segment_sum(msg, dst, num_segments=N)
        return agg + b_gcn

    x_flat = x_seq.reshape(B * T, N, F_IN)
    emb = jax.vmap(lambda xt: jnp.mean(jax.nn.relu(gcn(xt)), axis=0))(x_flat)
    emb_tbh = emb.reshape(B, T, H).transpose(1, 0, 2)
    return _lstm_head(emb_tbh, W_ih, W_hh, b_ih, b_hh, W_cls, b_cls)


# speedup vs baseline: 11.9385x; 11.9377x over previous
import functools

import jax
import jax.numpy as jnp
from jax import lax
from jax.experimental import pallas as pl
from jax.experimental.pallas import tpu as pltpu
from jax.experimental.pallas import tpu_sc as plsc

N = 10000
NPAD = 10240
E = 320000
F = 128
H = 128
B = 2
T = 8
K = B * T
ET = E + N
TR = 164
EPAD = TR * 128 * 16
DEG_CHUNK = EPAD // 32
NB = 5
NBK = N // NB
GARBAGE_ROW = N

_sc_mesh = plsc.VectorSubcoreMesh(
    core_axis_name="c", subcore_axis_name="s", num_cores=2, num_subcores=16)


@functools.partial(
    pl.kernel,
    out_type=jax.ShapeDtypeStruct((32, NPAD), jnp.float32),
    mesh=_sc_mesh,
    compiler_params=pltpu.CompilerParams(needs_layout_passes=False, use_tc_tiling_on_sc=False),
    scratch_types=[
        pltpu.VMEM((DEG_CHUNK,), jnp.int32),
        pltpu.VMEM((NPAD,), jnp.float32),
    ])
def _deg(dstv_hbm, out_hbm, dloc, degv):
    c = lax.axis_index("c")
    s = lax.axis_index("s")
    wid = s * 2 + c
    pltpu.sync_copy(dstv_hbm.at[pl.ds(wid * DEG_CHUNK, DEG_CHUNK)], dloc)

    def zbody(i, carry):
        degv[pl.ds(i * 16, 16)] = jnp.zeros((16,), jnp.float32)
        return carry

    lax.fori_loop(0, NPAD // 16, zbody, 0)
    ones = jnp.ones((16,), jnp.float32)

    def sbody(i, carry):
        idx = dloc[pl.ds(i * 16, 16)]
        plsc.addupdate_scatter(degv, [idx], ones)
        return carry

    lax.fori_loop(0, DEG_CHUNK // 16, sbody, 0)
    pltpu.sync_copy(degv, out_hbm.at[wid])


def _dinv_body(parts_ref, out_ref):
    parts = parts_ref[...]
    ones = jnp.ones((32, 1), jnp.float32)
    tot = lax.dot_general(parts, ones, (((0,), (0,)), ((), ())),
                          preferred_element_type=jnp.float32)
    out_ref[...] = lax.rsqrt(tot)


def _dinv(parts):
    return pl.pallas_call(
        _dinv_body,
        out_shape=jax.ShapeDtypeStruct((NPAD, 1), jnp.float32),
    )(parts)


def _table_body(x_ref, w_ref, dinv_ref, out_ref):
    hblk = jnp.dot(x_ref[0], w_ref[...], preferred_element_type=jnp.float32)
    out_ref[...] = hblk * dinv_ref[...]


def _table(x_tab, W_gcn, dinv):
    return pl.pallas_call(
        _table_body,
        grid=(K, NB),
        in_specs=[
            pl.BlockSpec((1, NBK, F), lambda k, nb: (k, nb, 0)),
            pl.BlockSpec((F, H), lambda k, nb: (0, 0)),
            pl.BlockSpec((NBK, 1), lambda k, nb: (nb, 0)),
        ],
        out_specs=pl.BlockSpec((NBK, H), lambda k, nb: (k * NB + nb, 0)),
        out_shape=jax.ShapeDtypeStruct((K * N, H), jnp.float32),
    )(x_tab, W_gcn, dinv)


@functools.partial(
    pl.kernel,
    out_type=jax.ShapeDtypeStruct((K, N, H), jnp.float32),
    mesh=_sc_mesh,
    compiler_params=pltpu.CompilerParams(needs_layout_passes=False, use_tc_tiling_on_sc=False),
    scratch_types=[
        pltpu.VMEM((128,), jnp.int32),
        pltpu.VMEM((128,), jnp.int32),
        pltpu.VMEM((128,), jnp.int32),
        pltpu.VMEM((128,), jnp.int32),
        pltpu.VMEM((128,), jnp.int32),
        pltpu.VMEM((128,), jnp.int32),
        pltpu.VMEM((128,), jnp.int32),
        pltpu.VMEM((128,), jnp.int32),
        pltpu.VMEM((128, H), jnp.float32),
        pltpu.VMEM((128, H), jnp.float32),
        pltpu.VMEM_SHARED((NPAD, H), jnp.float32),
        pltpu.SemaphoreType.DMA,
        pltpu.SemaphoreType.DMA,
        pltpu.SemaphoreType.DMA,
        pltpu.SemaphoreType.DMA,
        pltpu.SemaphoreType.DMA,
        pltpu.SemaphoreType.DMA,
        pltpu.SemaphoreType.DMA,
        pltpu.SemaphoreType.DMA,
        pltpu.SemaphoreType.DMA,
        pltpu.SemaphoreType.DMA,
    ])
def _seg(table_hbm, gsrc_hbm, dstv_hbm, zeros_hbm, agg_hbm,
         gb0, gb1, gb2, gb3, db0, db1, db2, db3, r0, r1, aggsh,
         si0, si1, si2, si3, sj0, sj1, sj2, sj3, sg0, sg1):
    c = lax.axis_index("c")
    s = lax.axis_index("s")
    base = s * (TR * 128)
    gb = [gb0, gb1, gb2, gb3]
    db = [db0, db1, db2, db3]
    rr = [r0, r1]
    si = [si0, si1, si2, si3]
    sj = [sj0, sj1, sj2, sj3]
    sg = [sg0, sg1]

    def fire_idx(b, j, k):
        pltpu.async_copy(gsrc_hbm.at[k, pl.ds(base + j * 128, 128)],
                         gb[b], si[b])
        pltpu.async_copy(dstv_hbm.at[pl.ds(base + j * 128, 128)],
                         db[b], sj[b])

    def wait_idx(b, k):
        pltpu.make_async_copy(gsrc_hbm.at[k, pl.ds(base, 128)],
                              gb[b], si[b]).wait()
        pltpu.make_async_copy(dstv_hbm.at[pl.ds(base, 128)],
                              db[b], sj[b]).wait()

    def fire_gather(b, rb):
        pltpu.async_copy(table_hbm.at[gb[b]], rr[rb], sg[rb])

    def wait_gather(b, rb):
        pltpu.make_async_copy(table_hbm.at[gb[b]], rr[rb], sg[rb]).wait()

    def scatter(b, rb):
        pltpu.sync_copy(rr[rb], aggsh.at[db[b]], add=True)

    for p in range(K // 2):
        k = c + 2 * p
        pltpu.sync_copy(zeros_hbm, aggsh.at[pl.ds(s * (NPAD // 16),
                                                  NPAD // 16)])
        for b in range(4):
            fire_idx(b, b, k)
        wait_idx(0, k)
        fire_gather(0, 0)
        wait_idx(1, k)
        fire_gather(1, 1)
        plsc.subcore_barrier()

        def body(i, carry):
            jo = 4 * i
            for b in range(4):
                rb = b % 2
                wait_gather(b, rb)
                scatter(b, rb)
                fire_idx(b, jo + b + 4, k)
                b2 = (b + 2) % 4
                wait_idx(b2, k)
                fire_gather(b2, rb)
            return carry

        lax.fori_loop(0, TR // 4 - 1, body, 0)
        wait_gather(0, 0)
        scatter(0, 0)
        wait_idx(2, k)
        fire_gather(2, 0)
        wait_gather(1, 1)
        scatter(1, 1)
        wait_idx(3, k)
        fire_gather(3, 1)
        wait_gather(2, 0)
        scatter(2, 0)
        wait_gather(3, 1)
        scatter(3, 1)
        plsc.subcore_barrier()
        pltpu.sync_copy(aggsh.at[pl.ds(s * (N // 16), N // 16)],
                        agg_hbm.at[k, pl.ds(s * (N // 16), N // 16)])
        plsc.subcore_barrier()


def _emb_body(agg_ref, dinv_ref, bg_ref, out_ref):
    nb = pl.program_id(1)
    a = agg_ref[0] * dinv_ref[...] + bg_ref[...]
    r = jnp.maximum(a, 0.0)
    part = jnp.sum(r, axis=0, keepdims=True).reshape(1, 1, H) * (1.0 / N)

    @pl.when(nb == 0)
    def _init():
        out_ref[...] = part

    @pl.when(nb > 0)
    def _acc():
        out_ref[...] = out_ref[...] + part


def _emb(agg, dinv, bg):
    return pl.pallas_call(
        _emb_body,
        grid=(K, NB),
        in_specs=[
            pl.BlockSpec((1, NBK, H), lambda k, nb: (k, nb, 0)),
            pl.BlockSpec((NBK, 1), lambda k, nb: (nb, 0)),
            pl.BlockSpec((1, H), lambda k, nb: (0, 0)),
        ],
        out_specs=pl.BlockSpec((1, 1, H), lambda k, nb: (k, 0, 0)),
        out_shape=jax.ShapeDtypeStruct((K, 1, H), jnp.float32),
    )(agg, dinv, bg)


def _head_body(emb_ref, wih_ref, whh_ref, bih_ref, bhh_ref, wcls_ref,
               bcls_ref, out_ref):
    wih = wih_ref[...]
    whh = whh_ref[...]
    b = bih_ref[...] + bhh_ref[...]

    def step(t, hc):
        h, cc = hc
        x_t = emb_ref[t]
        gates = (lax.dot_general(x_t, wih, (((1,), (1,)), ((), ())),
                                 preferred_element_type=jnp.float32)
                 + lax.dot_general(h, whh, (((1,), (1,)), ((), ())),
                                   preferred_element_type=jnp.float32)
                 + b)
        i = jax.nn.sigmoid(gates[:, 0 * H:1 * H])
        f = jax.nn.sigmoid(gates[:, 1 * H:2 * H])
        g = jnp.tanh(gates[:, 2 * H:3 * H])
        o = jax.nn.sigmoid(gates[:, 3 * H:4 * H])
        c_new = f * cc + i * g
        h_new = o * jnp.tanh(c_new)
        return (h_new, c_new)

    h0 = jnp.zeros((B, H), jnp.float32)
    c0 = jnp.zeros((B, H), jnp.float32)
    h, _ = lax.fori_loop(0, T, step, (h0, c0))
    out_ref[...] = jax.nn.sigmoid(
        jnp.dot(h, wcls_ref[...], preferred_element_type=jnp.float32)
        + bcls_ref[...])


def _head(emb_tbh, W_ih, W_hh, b_ih, b_hh, W_cls, b_cls):
    return pl.pallas_call(
        _head_body,
        out_shape=jax.ShapeDtypeStruct((B, 1), jnp.float32),
    )(emb_tbh, W_ih, W_hh, b_ih, b_hh, W_cls, b_cls)


def kernel(x_seq, edge_index, W_gcn, b_gcn, W_ih, W_hh, b_ih, b_hh, W_cls,
           b_cls):
    loopv = jnp.arange(N, dtype=jnp.int32)
    pad = EPAD - ET
    srcv = jnp.concatenate([edge_index[0].astype(jnp.int32), loopv,
                            jnp.zeros((pad,), jnp.int32)])
    dstv = jnp.concatenate([edge_index[1].astype(jnp.int32), loopv,
                            jnp.full((pad,), GARBAGE_ROW, jnp.int32)])
    gsrc = srcv[None, :] + (jnp.arange(K, dtype=jnp.int32) * N)[:, None]
    x_tab = x_seq.reshape(K, N, F)
    zeros_blk = jnp.zeros((NPAD // 16, H), jnp.float32)

    deg_parts = _deg(dstv)
    dinv = _dinv(deg_parts)
    table = _table(x_tab, W_gcn, dinv)
    agg = _seg(table, gsrc, dstv, zeros_blk)
    emb = _emb(agg, dinv, b_gcn.reshape(1, H)).reshape(K, H)
    emb_tbh = emb.reshape(B, T, H).transpose(1, 0, 2)
    return _head(emb_tbh, W_ih, W_hh, b_ih.reshape(1, 4 * H),
                 b_hh.reshape(1, 4 * H), W_cls, b_cls.reshape(1, 1))
